# baseline (speedup 1.0000x reference)
import jax
import jax.numpy as jnp
from jax.experimental import pallas as pl
from jax.experimental.pallas import tpu as pltpu

NB_ANGLES = 100
B = 32
N = 8192
SUB = 64
A_BLK = 4
ROWS = 2 * SUB


N_MXU_CHAINS = 2


def _xor_partner_mxu(w, p_ref, j):
    if j >= 128:
        e = (j >> 7).bit_length() - 1
        pm = p_ref[e]
        return jax.lax.dot_general(
            pm, w, (((1,), (0,)), ((), ())),
            precision=jax.lax.Precision.HIGHEST,
            preferred_element_type=jnp.float32)
    e = j.bit_length() - 1
    pm = p_ref[e]
    return jax.lax.dot_general(
        w, pm, (((1,), (0,)), ((), ())),
        precision=jax.lax.Precision.HIGHEST,
        preferred_element_type=jnp.float32)


def _xor_partner_xlu(w, jb, j):
    if j >= 128:
        m = j >> 7
        up = pltpu.roll(w, ROWS - m, 0)
        dn = pltpu.roll(w, m, 0)
    else:
        up = pltpu.roll(w, 128 - j, 1)
        dn = pltpu.roll(w, j, 1)
    return jnp.where(jb, up, dn)


def _bitonic_sort_chains(ys, p_ref):
    shape = ys[0].shape
    s_iota = jax.lax.broadcasted_iota(jnp.int32, shape, 0)
    l_iota = jax.lax.broadcasted_iota(jnp.int32, shape, 1)
    idx = ((s_iota & (SUB - 1)) << 7) | l_iota

    one = jnp.float32(1.0)
    neg = jnp.float32(-1.0)

    for st in range(1, 14):
        k = 1 << st
        if k < N:
            sgn = jnp.where((idx & k) == 0, one, neg)
            ws = [y * sgn for y in ys]
        else:
            ws = ys
        for e in range(st - 1, -1, -1):
            j = 1 << e
            jb = (idx & j) == 0
            for c in range(len(ws)):
                w = ws[c]
                if c < N_MXU_CHAINS:
                    partner = _xor_partner_mxu(w, p_ref, j)
                else:
                    partner = _xor_partner_xlu(w, jb, j)
                ws[c] = jnp.where(jb, jnp.minimum(w, partner),
                                  jnp.maximum(w, partner))
        if k < N:
            ys = [w * sgn for w in ws]
        else:
            ys = ws
    return ys


def _swd_kernel(ang_ref, proba_ref, p_ref, x_ref, t_ref, out_ref, acc_ref):
    b = pl.program_id(0)
    ab = pl.program_id(1)
    na = pl.num_programs(1)

    ys = []
    for a in range(A_BLK):
        aidx = ab * A_BLK + a
        a0 = ang_ref[0, aidx]
        a1 = ang_ref[1, aidx]
        a2 = ang_ref[2, aidx]
        pin = x_ref[0, 0] * a0 + x_ref[0, 1] * a1 + x_ref[0, 2] * a2
        ptg = t_ref[0, 0] * a0 + t_ref[0, 1] * a1 + t_ref[0, 2] * a2
        ys.append(jnp.concatenate([pin, ptg], axis=0))

    ys = _bitonic_sort_chains(ys, p_ref)

    ssum = jnp.float32(0.0)
    for y in ys:
        ssum += jnp.sum(jnp.abs(y[:SUB] - y[SUB:]))

    pw = proba_ref[b]

    @pl.when(jnp.logical_and(b == 0, ab == 0))
    def _():
        acc_ref[0] = 0.0
        acc_ref[1] = 0.0

    acc_ref[0] += pw * ssum * (1.0 / (N * NB_ANGLES))

    @pl.when(ab == 0)
    def _():
        acc_ref[1] += pw

    @pl.when(jnp.logical_and(b == B - 1, ab == na - 1))
    def _():
        out_ref[0] = acc_ref[0] / acc_ref[1]


@jax.jit
def kernel(input, target, proba):
    akey = jax.random.key(42)
    angles = jax.random.uniform(akey, (1, 1, NB_ANGLES, 3), dtype=jnp.float32) * 2.0 - 1.0
    angles = angles / jnp.linalg.norm(angles, axis=-1, keepdims=True)
    ang = angles.reshape(NB_ANGLES, 3).T

    x = input.transpose(0, 2, 1).reshape(B, 3, SUB, 128)
    t = target.transpose(0, 2, 1).reshape(B, 3, SUB, 128)

    lane = jnp.arange(128, dtype=jnp.int32)
    pmats = jnp.stack([
        ((lane[:, None] ^ (1 << e)) == lane[None, :]).astype(jnp.float32)
        for e in range(7)
    ])

    grid = (B, NB_ANGLES // A_BLK)
    out = pl.pallas_call(
        _swd_kernel,
        grid=grid,
        in_specs=[
            pl.BlockSpec(memory_space=pltpu.SMEM),
            pl.BlockSpec(memory_space=pltpu.SMEM),
            pl.BlockSpec((7, 128, 128), lambda b, a: (0, 0, 0)),
            pl.BlockSpec((1, 3, SUB, 128), lambda b, a: (b, 0, 0, 0)),
            pl.BlockSpec((1, 3, SUB, 128), lambda b, a: (b, 0, 0, 0)),
        ],
        out_specs=pl.BlockSpec(memory_space=pltpu.SMEM),
        out_shape=jax.ShapeDtypeStruct((1,), jnp.float32),
        scratch_shapes=[pltpu.SMEM((2,), jnp.float32)],
    )(ang, proba, pmats, x, t)
    return out.reshape(())


# speedup vs baseline: 3.3866x; 3.3866x over previous
import jax
import jax.numpy as jnp
from jax.experimental import pallas as pl
from jax.experimental.pallas import tpu as pltpu

NB_ANGLES = 100
B = 32
N = 8192
SUB = 64
A_BLK = 4
ROWS = 2 * SUB
PAD = 32


def _bitonic_sort_chains(ys, wbuf_ref):
    shape = ys[0].shape
    s_iota = jax.lax.broadcasted_iota(jnp.int32, shape, 0)
    l_iota = jax.lax.broadcasted_iota(jnp.int32, shape, 1)
    idx = (l_iota << 6) | (s_iota & (SUB - 1))

    one = jnp.float32(1.0)
    neg = jnp.float32(-1.0)

    for st in range(1, 14):
        k = 1 << st
        if k < N:
            sgn = jnp.where((idx & k) == 0, one, neg)
            ws = [y * sgn for y in ys]
        else:
            ws = ys
        for e in range(st - 1, -1, -1):
            j = 1 << e
            jb = (idx & j) == 0
            for c in range(len(ws)):
                w = ws[c]
                if j < SUB:
                    wbuf_ref[c, pl.ds(PAD, ROWS), :] = w
                    up = wbuf_ref[c, pl.ds(PAD + j, ROWS), :]
                    dn = wbuf_ref[c, pl.ds(PAD - j, ROWS), :]
                else:
                    q = j >> 6
                    up = pltpu.roll(w, 128 - q, 1)
                    dn = pltpu.roll(w, q, 1)
                partner = jnp.where(jb, up, dn)
                ws[c] = jnp.where(jb, jnp.minimum(w, partner),
                                  jnp.maximum(w, partner))
        if k < N:
            ys = [w * sgn for w in ws]
        else:
            ys = ws
    return ys


def _swd_kernel(ang_ref, proba_ref, x_ref, t_ref, out_ref, wbuf_ref, acc_ref):
    b = pl.program_id(0)
    ab = pl.program_id(1)
    na = pl.num_programs(1)

    ys = []
    for a in range(A_BLK):
        aidx = ab * A_BLK + a
        a0 = ang_ref[0, aidx]
        a1 = ang_ref[1, aidx]
        a2 = ang_ref[2, aidx]
        pin = x_ref[0, 0] * a0 + x_ref[0, 1] * a1 + x_ref[0, 2] * a2
        ptg = t_ref[0, 0] * a0 + t_ref[0, 1] * a1 + t_ref[0, 2] * a2
        ys.append(jnp.concatenate([pin, ptg], axis=0))

    ys = _bitonic_sort_chains(ys, wbuf_ref)

    ssum = jnp.float32(0.0)
    for y in ys:
        ssum += jnp.sum(jnp.abs(y[:SUB] - y[SUB:]))

    pw = proba_ref[b]

    @pl.when(jnp.logical_and(b == 0, ab == 0))
    def _():
        acc_ref[0] = 0.0
        acc_ref[1] = 0.0

    acc_ref[0] += pw * ssum * (1.0 / (N * NB_ANGLES))

    @pl.when(ab == 0)
    def _():
        acc_ref[1] += pw

    @pl.when(jnp.logical_and(b == B - 1, ab == na - 1))
    def _():
        out_ref[0] = acc_ref[0] / acc_ref[1]


@jax.jit
def kernel(input, target, proba):
    akey = jax.random.key(42)
    angles = jax.random.uniform(akey, (1, 1, NB_ANGLES, 3), dtype=jnp.float32) * 2.0 - 1.0
    angles = angles / jnp.linalg.norm(angles, axis=-1, keepdims=True)
    ang = angles.reshape(NB_ANGLES, 3).T

    x = input.transpose(0, 2, 1).reshape(B, 3, 128, SUB).transpose(0, 1, 3, 2)
    t = target.transpose(0, 2, 1).reshape(B, 3, 128, SUB).transpose(0, 1, 3, 2)

    grid = (B, NB_ANGLES // A_BLK)
    out = pl.pallas_call(
        _swd_kernel,
        grid=grid,
        in_specs=[
            pl.BlockSpec(memory_space=pltpu.SMEM),
            pl.BlockSpec(memory_space=pltpu.SMEM),
            pl.BlockSpec((1, 3, SUB, 128), lambda b, a: (b, 0, 0, 0)),
            pl.BlockSpec((1, 3, SUB, 128), lambda b, a: (b, 0, 0, 0)),
        ],
        out_specs=pl.BlockSpec(memory_space=pltpu.SMEM),
        out_shape=jax.ShapeDtypeStruct((1,), jnp.float32),
        scratch_shapes=[
            pltpu.VMEM((A_BLK, ROWS + 2 * PAD, 128), jnp.float32),
            pltpu.SMEM((2,), jnp.float32),
        ],
    )(ang, proba, x, t)
    return out.reshape(())


# speedup vs baseline: 3.6575x; 1.0800x over previous
import jax
import jax.numpy as jnp
from jax.experimental import pallas as pl
from jax.experimental.pallas import tpu as pltpu

NB_ANGLES = 100
B = 32
N = 8192
SUB = 64
A_BLK = 4
ROWS = 2 * SUB
PAD = 32


def _bitonic_sort_chains(ys, wbuf_ref):
    shape = ys[0].shape
    s_iota = jax.lax.broadcasted_iota(jnp.int32, shape, 0)
    l_iota = jax.lax.broadcasted_iota(jnp.int32, shape, 1)
    idx = (l_iota << 6) | (s_iota & (SUB - 1))

    one = jnp.float32(1.0)
    neg = jnp.float32(-1.0)

    for st in range(1, 14):
        k = 1 << st
        if k < N:
            sgn = jnp.where((idx & k) == 0, one, neg)
            ws = [y * sgn for y in ys]
        else:
            ws = ys
        for e in range(st - 1, -1, -1):
            j = 1 << e
            jb = (idx & j) == 0
            for c in range(len(ws)):
                w = ws[c]
                if j < SUB:
                    wbuf_ref[c, pl.ds(PAD, ROWS), :] = w
                    up = wbuf_ref[c, pl.ds(PAD + j, ROWS), :]
                    dn = wbuf_ref[c, pl.ds(PAD - j, ROWS), :]
                else:
                    q = j >> 6
                    up = pltpu.roll(w, 128 - q, 1)
                    dn = pltpu.roll(w, q, 1)
                ws[c] = jnp.where(jb, jnp.minimum(w, up), jnp.maximum(w, dn))
        if k < N:
            ys = [w * sgn for w in ws]
        else:
            ys = ws
    return ys


def _swd_kernel(ang_ref, proba_ref, x_ref, t_ref, out_ref, wbuf_ref, acc_ref):
    b = pl.program_id(0)
    ab = pl.program_id(1)
    na = pl.num_programs(1)

    ys = []
    for a in range(A_BLK):
        aidx = ab * A_BLK + a
        a0 = ang_ref[0, aidx]
        a1 = ang_ref[1, aidx]
        a2 = ang_ref[2, aidx]
        pin = x_ref[0, 0] * a0 + x_ref[0, 1] * a1 + x_ref[0, 2] * a2
        ptg = t_ref[0, 0] * a0 + t_ref[0, 1] * a1 + t_ref[0, 2] * a2
        ys.append(jnp.concatenate([pin, ptg], axis=0))

    ys = _bitonic_sort_chains(ys, wbuf_ref)

    ssum = jnp.float32(0.0)
    for y in ys:
        ssum += jnp.sum(jnp.abs(y[:SUB] - y[SUB:]))

    pw = proba_ref[b]

    @pl.when(jnp.logical_and(b == 0, ab == 0))
    def _():
        acc_ref[0] = 0.0
        acc_ref[1] = 0.0

    acc_ref[0] += pw * ssum * (1.0 / (N * NB_ANGLES))

    @pl.when(ab == 0)
    def _():
        acc_ref[1] += pw

    @pl.when(jnp.logical_and(b == B - 1, ab == na - 1))
    def _():
        out_ref[0] = acc_ref[0] / acc_ref[1]


@jax.jit
def kernel(input, target, proba):
    akey = jax.random.key(42)
    angles = jax.random.uniform(akey, (1, 1, NB_ANGLES, 3), dtype=jnp.float32) * 2.0 - 1.0
    angles = angles / jnp.linalg.norm(angles, axis=-1, keepdims=True)
    ang = angles.reshape(NB_ANGLES, 3).T

    x = input.transpose(0, 2, 1).reshape(B, 3, 128, SUB).transpose(0, 1, 3, 2)
    t = target.transpose(0, 2, 1).reshape(B, 3, 128, SUB).transpose(0, 1, 3, 2)

    grid = (B, NB_ANGLES // A_BLK)
    out = pl.pallas_call(
        _swd_kernel,
        grid=grid,
        in_specs=[
            pl.BlockSpec(memory_space=pltpu.SMEM),
            pl.BlockSpec(memory_space=pltpu.SMEM),
            pl.BlockSpec((1, 3, SUB, 128), lambda b, a: (b, 0, 0, 0)),
            pl.BlockSpec((1, 3, SUB, 128), lambda b, a: (b, 0, 0, 0)),
        ],
        out_specs=pl.BlockSpec(memory_space=pltpu.SMEM),
        out_shape=jax.ShapeDtypeStruct((1,), jnp.float32),
        scratch_shapes=[
            pltpu.VMEM((A_BLK, ROWS + 2 * PAD, 128), jnp.float32),
            pltpu.SMEM((2,), jnp.float32),
        ],
    )(ang, proba, x, t)
    return out.reshape(())


# speedup vs baseline: 4.1236x; 1.1274x over previous
import jax
import jax.numpy as jnp
from jax.experimental import pallas as pl
from jax.experimental.pallas import tpu as pltpu

NB_ANGLES = 100
B = 32
N = 8192
SUB = 64
A_BLK = 10
ROWS = 2 * SUB
PAD = 32


def _bitonic_sort_vmem(wbuf_ref, idx):
    one = jnp.float32(1.0)
    neg = jnp.float32(-1.0)

    def sgn_of(st):
        k = 1 << st
        if k >= N:
            return None
        return jnp.where((idx & k) == 0, one, neg)

    for st in range(1, 14):
        for e in range(st - 1, -1, -1):
            j = 1 << e
            jb = (idx & j) == 0
            cf = None
            if e == 0 and st < 13:
                s_cur, s_nxt = sgn_of(st), sgn_of(st + 1)
                cf = s_cur if s_nxt is None else s_cur * s_nxt
            for c in range(A_BLK):
                w = wbuf_ref[c, pl.ds(PAD, ROWS), :]
                if j < SUB:
                    up = wbuf_ref[c, pl.ds(PAD + j, ROWS), :]
                    dn = wbuf_ref[c, pl.ds(PAD - j, ROWS), :]
                else:
                    q = j >> 6
                    up = pltpu.roll(w, 128 - q, 1)
                    dn = pltpu.roll(w, q, 1)
                wnew = jnp.where(jb, jnp.minimum(w, up), jnp.maximum(w, dn))
                if cf is not None:
                    wnew = wnew * cf
                wbuf_ref[c, pl.ds(PAD, ROWS), :] = wnew


def _swd_kernel(ang_ref, proba_ref, x_ref, t_ref, out_ref, wbuf_ref, acc_ref):
    b = pl.program_id(0)
    ab = pl.program_id(1)
    na = pl.num_programs(1)

    shape = (ROWS, 128)
    s_iota = jax.lax.broadcasted_iota(jnp.int32, shape, 0)
    l_iota = jax.lax.broadcasted_iota(jnp.int32, shape, 1)
    idx = (l_iota << 6) | (s_iota & (SUB - 1))

    for a in range(A_BLK):
        aidx = ab * A_BLK + a
        a0 = ang_ref[0, aidx]
        a1 = ang_ref[1, aidx]
        a2 = ang_ref[2, aidx]
        pin = x_ref[0, 0] * a0 + x_ref[0, 1] * a1 + x_ref[0, 2] * a2
        ptg = t_ref[0, 0] * a0 + t_ref[0, 1] * a1 + t_ref[0, 2] * a2
        sgn1 = jnp.where((idx & 2) == 0, jnp.float32(1.0), jnp.float32(-1.0))
        wbuf_ref[a, pl.ds(PAD, ROWS), :] = jnp.concatenate([pin, ptg], axis=0) * sgn1

    _bitonic_sort_vmem(wbuf_ref, idx)

    ssum = jnp.float32(0.0)
    for a in range(A_BLK):
        y = wbuf_ref[a, pl.ds(PAD, ROWS), :]
        ssum += jnp.sum(jnp.abs(y[:SUB] - y[SUB:]))

    pw = proba_ref[b]

    @pl.when(jnp.logical_and(b == 0, ab == 0))
    def _():
        acc_ref[0] = 0.0
        acc_ref[1] = 0.0

    acc_ref[0] += pw * ssum * (1.0 / (N * NB_ANGLES))

    @pl.when(ab == 0)
    def _():
        acc_ref[1] += pw

    @pl.when(jnp.logical_and(b == B - 1, ab == na - 1))
    def _():
        out_ref[0] = acc_ref[0] / acc_ref[1]


@jax.jit
def kernel(input, target, proba):
    akey = jax.random.key(42)
    angles = jax.random.uniform(akey, (1, 1, NB_ANGLES, 3), dtype=jnp.float32) * 2.0 - 1.0
    angles = angles / jnp.linalg.norm(angles, axis=-1, keepdims=True)
    ang = angles.reshape(NB_ANGLES, 3).T

    x = input.transpose(0, 2, 1).reshape(B, 3, 128, SUB).transpose(0, 1, 3, 2)
    t = target.transpose(0, 2, 1).reshape(B, 3, 128, SUB).transpose(0, 1, 3, 2)

    grid = (B, NB_ANGLES // A_BLK)
    out = pl.pallas_call(
        _swd_kernel,
        grid=grid,
        in_specs=[
            pl.BlockSpec(memory_space=pltpu.SMEM),
            pl.BlockSpec(memory_space=pltpu.SMEM),
            pl.BlockSpec((1, 3, SUB, 128), lambda b, a: (b, 0, 0, 0)),
            pl.BlockSpec((1, 3, SUB, 128), lambda b, a: (b, 0, 0, 0)),
        ],
        out_specs=pl.BlockSpec(memory_space=pltpu.SMEM),
        out_shape=jax.ShapeDtypeStruct((1,), jnp.float32),
        scratch_shapes=[
            pltpu.VMEM((A_BLK, ROWS + 2 * PAD, 128), jnp.float32),
            pltpu.SMEM((2,), jnp.float32),
        ],
    )(ang, proba, x, t)
    return out.reshape(())
